# baseline (speedup 1.0000x reference)
import functools

import jax
import jax.numpy as jnp
from jax import lax
from jax.experimental import pallas as pl
from jax.experimental.pallas import tpu as pltpu
from jax.experimental.pallas import tpu_sc as plsc

N = 10000
NPAD = 10240
F = 256
E = 160000
HALF = F // 2
L = 16
NS = 16
EPT = E // NS
CHUNK = 80
NCHUNK = EPT // CHUNK
RPT = NPAD // NS
BN = 400


def _proj_body(h_ref, qwt_ref, qb_ref, o_ref):
    y = jnp.dot(h_ref[...], qwt_ref[...], preferred_element_type=jnp.float32)
    y = y + qb_ref[...]
    o_ref[...] = jnp.where(y > 0, y, 0.01 * y)


def _project(h, qwt, qb2):
    return pl.pallas_call(
        _proj_body,
        grid=(2, N // BN),
        in_specs=[
            pl.BlockSpec((BN, F), lambda p, i: (i, 0)),
            pl.BlockSpec((F, HALF), lambda p, i: (0, p)),
            pl.BlockSpec((1, HALF), lambda p, i: (0, p)),
        ],
        out_specs=pl.BlockSpec((BN, HALF), lambda p, i: (p * (N // BN) + i, 0)),
        out_shape=jax.ShapeDtypeStruct((2 * N, HALF), jnp.float32),
    )(h, qwt, qb2)


def _apply_body(h_ref, lo_ref, hi_ref, w_ref, a_ref, b1_ref, b2_ref, wb_ref,
                o_ref):
    w = w_ref[:, 0:1]
    w = jnp.where(w == 0.0, 1.0, w)
    y = jnp.dot(h_ref[...], a_ref[...], preferred_element_type=jnp.float32)
    agg = jnp.dot(lo_ref[...], b1_ref[...], preferred_element_type=jnp.float32)
    agg = agg + jnp.dot(hi_ref[...], b2_ref[...],
                        preferred_element_type=jnp.float32)
    y = y + agg / w + wb_ref[...]
    y = jnp.where(y > 0, y, 0.01 * y)
    nrm = jnp.sqrt(jnp.sum(y * y, axis=1, keepdims=True))
    nrm = jnp.where(nrm == 0.0, 1.0, nrm)
    o_ref[...] = y / nrm


def _apply(h, lo, hi, w16, a, b1, b2, wb2):
    return pl.pallas_call(
        _apply_body,
        grid=(N // BN,),
        in_specs=[
            pl.BlockSpec((BN, F), lambda i: (i, 0)),
            pl.BlockSpec((BN, HALF), lambda i: (i, 0)),
            pl.BlockSpec((BN, HALF), lambda i: (i, 0)),
            pl.BlockSpec((BN, L), lambda i: (i, 0)),
            pl.BlockSpec((F, F), lambda i: (0, 0)),
            pl.BlockSpec((HALF, F), lambda i: (0, 0)),
            pl.BlockSpec((HALF, F), lambda i: (0, 0)),
            pl.BlockSpec((1, F), lambda i: (0, 0)),
        ],
        out_specs=pl.BlockSpec((BN, F), lambda i: (i, 0)),
        out_shape=jax.ShapeDtypeStruct((N, F), jnp.float32),
    )(h, lo, hi, w16, a, b1, b2, wb2)


def _apply_proj_body(h_ref, lo_ref, hi_ref, w_ref, a_ref, b1_ref, b2_ref,
                     wb_ref, qwt_ref, qb_ref, o_ref, t_ref):
    w = w_ref[:, 0:1]
    w = jnp.where(w == 0.0, 1.0, w)
    y = jnp.dot(h_ref[...], a_ref[...], preferred_element_type=jnp.float32)
    agg = jnp.dot(lo_ref[...], b1_ref[...], preferred_element_type=jnp.float32)
    agg = agg + jnp.dot(hi_ref[...], b2_ref[...],
                        preferred_element_type=jnp.float32)
    y = y + agg / w + wb_ref[...]
    y = jnp.where(y > 0, y, 0.01 * y)
    nrm = jnp.sqrt(jnp.sum(y * y, axis=1, keepdims=True))
    nrm = jnp.where(nrm == 0.0, 1.0, nrm)
    y = y / nrm
    o_ref[...] = y
    t = jnp.dot(y, qwt_ref[...], preferred_element_type=jnp.float32)
    t = t + qb_ref[...]
    t = jnp.where(t > 0, t, 0.01 * t)
    t_ref[0] = t[:, :HALF]
    t_ref[1] = t[:, HALF:]


def _apply_project(h, lo, hi, w16, a, b1, b2, wb2, qwt, qb2):
    return pl.pallas_call(
        _apply_proj_body,
        grid=(N // BN,),
        in_specs=[
            pl.BlockSpec((BN, F), lambda i: (i, 0)),
            pl.BlockSpec((BN, HALF), lambda i: (i, 0)),
            pl.BlockSpec((BN, HALF), lambda i: (i, 0)),
            pl.BlockSpec((BN, L), lambda i: (i, 0)),
            pl.BlockSpec((F, F), lambda i: (0, 0)),
            pl.BlockSpec((HALF, F), lambda i: (0, 0)),
            pl.BlockSpec((HALF, F), lambda i: (0, 0)),
            pl.BlockSpec((1, F), lambda i: (0, 0)),
            pl.BlockSpec((F, F), lambda i: (0, 0)),
            pl.BlockSpec((1, F), lambda i: (0, 0)),
        ],
        out_specs=[
            pl.BlockSpec((BN, F), lambda i: (i, 0)),
            pl.BlockSpec((2, BN, HALF), lambda i: (0, i, 0)),
        ],
        out_shape=[
            jax.ShapeDtypeStruct((N, F), jnp.float32),
            jax.ShapeDtypeStruct((2, N, HALF), jnp.float32),
        ],
    )(h, lo, hi, w16, a, b1, b2, wb2, qwt, qb2)


def _agg_body(table, src2, dst, ppr, out_lo, out_hi, out_w, srcb, dstb,
              gbuf, pprb, wbuf, acc, wsh, gsem0, gsem1, gsem2, ssem0, ssem1,
              ssem2, psem0, psem1, psem2, wsem0, wsem1, wsem2):
    c = lax.axis_index("c")
    s = lax.axis_index("s")
    ebase = pl.multiple_of(c * E + s * EPT, 8)
    rbase = pl.multiple_of(s * RPT, 8)

    gsem = (gsem0, gsem1, gsem2)
    ssem = (ssem0, ssem1, ssem2)
    psem = (psem0, psem1, psem2)
    wsem = (wsem0, wsem1, wsem2)

    def _zb(i, carry):
        for k in range(HALF // L):
            gbuf[0, i, pl.ds(L * k, L)] = jnp.zeros((L,), jnp.float32)
        wbuf[0, i, :] = jnp.zeros((L,), jnp.float32)
        return carry

    lax.fori_loop(0, CHUNK, _zb, 0)
    for t in range(RPT // CHUNK):
        sl = pl.ds(rbase + t * CHUNK, CHUNK)
        pltpu.sync_copy(gbuf.at[0], acc.at[sl])

        @pl.when(c == 0)
        def _():
            pltpu.sync_copy(wbuf.at[0], wsh.at[sl])

    plsc.subcore_barrier()

    def _idx_issue(j, b):
        base = pl.multiple_of(ebase + j * CHUNK, 8)
        pltpu.async_copy(src2.at[pl.ds(base, CHUNK)], srcb.at[b], psem[b])
        bd = pl.multiple_of((ebase - c * E) + j * CHUNK, 8)
        pltpu.async_copy(dst.at[pl.ds(bd, CHUNK)], dstb.at[b], psem[b])
        pltpu.async_copy(ppr.at[pl.ds(bd, CHUNK)], pprb.at[b], psem[b])

    def _gather_issue(j, b):
        pltpu.make_async_copy(src2.at[pl.ds(0, CHUNK)], srcb.at[b],
                              psem[b]).wait()
        pltpu.make_async_copy(dst.at[pl.ds(0, CHUNK)], dstb.at[b],
                              psem[b]).wait()
        pltpu.make_async_copy(ppr.at[pl.ds(0, CHUNK)], pprb.at[b],
                              psem[b]).wait()
        pltpu.async_copy(table.at[srcb.at[b]], gbuf.at[b], gsem[b])

    def _wait_scatter(b):
        pltpu.make_async_copy(gbuf.at[b], acc.at[dstb.at[b]], ssem[b]).wait()

        @pl.when(c == 0)
        def _():
            pltpu.make_async_copy(wbuf.at[b], wsh.at[dstb.at[b]],
                                  wsem[b]).wait()

    def _process(j, b, first=False, last=0):
        bp = (b + 2) % 3
        bn = (b + 1) % 3
        pltpu.make_async_copy(table.at[srcb.at[b]], gbuf.at[b],
                              gsem[b]).wait()

        def _grp(g, rc):
            pv = pprb[b, pl.ds(g * L, L)]
            for r in range(L):
                spl = jnp.broadcast_to(pv[r], (L,))
                row = g * L + r
                for k in range(HALF // L):
                    sl = pl.ds(L * k, L)
                    gbuf[b, row, sl] = gbuf[b, row, sl] * spl
                wbuf[b, row, :] = spl
            return rc

        lax.fori_loop(0, CHUNK // L, _grp, 0)
        pltpu.async_copy(gbuf.at[b], acc.at[dstb.at[b]], ssem[b], add=True)
        if not first:
            _wait_scatter(bp)
        if last < 2:
            _idx_issue(j + 2, bp)
        if last < 3:
            _gather_issue(j + 1, bn)

        @pl.when(c == 0)
        def _():
            pltpu.async_copy(wbuf.at[b], wsh.at[dstb.at[b]], wsem[b],
                             add=True)

    _idx_issue(0, 0)
    _idx_issue(1, 1)
    _gather_issue(0, 0)
    _process(0, 0, first=True)
    _process(1, 1)
    _process(2, 2)

    def _trip(t, carry):
        j = 3 * t
        _process(j, 0)
        _process(j + 1, 1)
        _process(j + 2, 2)
        return carry

    lax.fori_loop(1, (NCHUNK - 2) // 3, _trip, 0)
    _process(NCHUNK - 2, 0, last=2)
    _process(NCHUNK - 1, 1, last=3)
    _wait_scatter(1)
    plsc.subcore_barrier()

    @pl.when(c == 0)
    def _():
        pltpu.sync_copy(acc.at[pl.ds(rbase, RPT)], out_lo.at[pl.ds(rbase, RPT)])
        pltpu.sync_copy(wsh.at[pl.ds(rbase, RPT)], out_w.at[pl.ds(rbase, RPT)])

    @pl.when(c == 1)
    def _():
        pltpu.sync_copy(acc.at[pl.ds(rbase, RPT)], out_hi.at[pl.ds(rbase, RPT)])


@functools.partial(jax.jit, static_argnames=())
def _aggregate(table, src2, dst, ppr):
    mesh = plsc.VectorSubcoreMesh(core_axis_name="c", subcore_axis_name="s")
    return pl.kernel(
        _agg_body,
        out_type=[
            jax.ShapeDtypeStruct((NPAD, HALF), jnp.float32),
            jax.ShapeDtypeStruct((NPAD, HALF), jnp.float32),
            jax.ShapeDtypeStruct((NPAD, L), jnp.float32),
        ],
        mesh=mesh,
        scratch_types=[
            pltpu.VMEM((3, CHUNK), jnp.int32),
            pltpu.VMEM((3, CHUNK), jnp.int32),
            pltpu.VMEM((3, CHUNK, HALF), jnp.float32),
            pltpu.VMEM((3, CHUNK), jnp.float32),
            pltpu.VMEM((3, CHUNK, L), jnp.float32),
            pltpu.VMEM_SHARED((NPAD, HALF), jnp.float32),
            pltpu.VMEM_SHARED((NPAD, L), jnp.float32),
        ] + [pltpu.SemaphoreType.DMA] * 12,
        compiler_params=pltpu.CompilerParams(use_tc_tiling_on_sc=False),
    )(table, src2, dst, ppr)


def kernel(x, edge_index, ppr_weight, Q0_w, Q0_b, W0_w, W0_b, Q1_w, Q1_b,
           W1_w, W1_b):
    src = edge_index[0]
    dst = edge_index[1]
    src2 = jnp.concatenate([src, src + N])

    def wparts(ww, wb):
        wwt = ww.T
        return wwt[:F], wwt[F:F + HALF], wwt[F + HALF:], wb.reshape(1, F)

    a0, b10, b20, wb0 = wparts(W0_w, W0_b)
    a1, b11, b21, wb1 = wparts(W1_w, W1_b)

    table = _project(x, Q0_w.T, Q0_b.reshape(1, F))
    lo, hi, w16 = _aggregate(table, src2, dst, ppr_weight)
    h1, table3 = _apply_project(x, lo, hi, w16, a0, b10, b20, wb0,
                                Q1_w.T, Q1_b.reshape(1, F))
    lo, hi, w16 = _aggregate(table3.reshape(2 * N, HALF), src2, dst,
                             ppr_weight)
    return _apply(h1, lo, hi, w16, a1, b11, b21, wb1)


# speedup vs baseline: 1.3040x; 1.3040x over previous
import functools

import jax
import jax.numpy as jnp
from jax import lax
from jax.experimental import pallas as pl
from jax.experimental.pallas import tpu as pltpu
from jax.experimental.pallas import tpu_sc as plsc

N = 10000
NPAD = 10240
F = 256
E = 160000
HALF = F // 2
L = 16
NS = 16
EPT = E // NS
CHUNK = 80
NCHUNK = EPT // CHUNK
RPT = NPAD // NS
BN = 400


def _proj_body(h_ref, qwt_ref, qb_ref, o_ref):
    y = jnp.dot(h_ref[...], qwt_ref[...], preferred_element_type=jnp.float32)
    y = y + qb_ref[...]
    o_ref[...] = jnp.where(y > 0, y, 0.01 * y)


def _project(h, qwt, qb2):
    return pl.pallas_call(
        _proj_body,
        grid=(2, N // BN),
        in_specs=[
            pl.BlockSpec((BN, F), lambda p, i: (i, 0)),
            pl.BlockSpec((F, HALF), lambda p, i: (0, p)),
            pl.BlockSpec((1, HALF), lambda p, i: (0, p)),
        ],
        out_specs=pl.BlockSpec((BN, HALF), lambda p, i: (p * (N // BN) + i, 0)),
        out_shape=jax.ShapeDtypeStruct((2 * N, HALF), jnp.float32),
    )(h, qwt, qb2)


def _apply_body(h_ref, lo_ref, hi_ref, w_ref, a_ref, b1_ref, b2_ref, wb_ref,
                o_ref):
    w = w_ref[:, 0:1]
    w = jnp.where(w == 0.0, 1.0, w)
    y = jnp.dot(h_ref[...], a_ref[...], preferred_element_type=jnp.float32)
    agg = jnp.dot(lo_ref[...], b1_ref[...], preferred_element_type=jnp.float32)
    agg = agg + jnp.dot(hi_ref[...], b2_ref[...],
                        preferred_element_type=jnp.float32)
    y = y + agg / w + wb_ref[...]
    y = jnp.where(y > 0, y, 0.01 * y)
    nrm = jnp.sqrt(jnp.sum(y * y, axis=1, keepdims=True))
    nrm = jnp.where(nrm == 0.0, 1.0, nrm)
    o_ref[...] = y / nrm


def _apply(h, lo, hi, w16, a, b1, b2, wb2):
    return pl.pallas_call(
        _apply_body,
        grid=(N // BN,),
        in_specs=[
            pl.BlockSpec((BN, F), lambda i: (i, 0)),
            pl.BlockSpec((BN, HALF), lambda i: (i, 0)),
            pl.BlockSpec((BN, HALF), lambda i: (i, 0)),
            pl.BlockSpec((BN, L), lambda i: (i, 0)),
            pl.BlockSpec((F, F), lambda i: (0, 0)),
            pl.BlockSpec((HALF, F), lambda i: (0, 0)),
            pl.BlockSpec((HALF, F), lambda i: (0, 0)),
            pl.BlockSpec((1, F), lambda i: (0, 0)),
        ],
        out_specs=pl.BlockSpec((BN, F), lambda i: (i, 0)),
        out_shape=jax.ShapeDtypeStruct((N, F), jnp.float32),
    )(h, lo, hi, w16, a, b1, b2, wb2)


def _apply_proj_body(h_ref, lo_ref, hi_ref, w_ref, a_ref, b1_ref, b2_ref,
                     wb_ref, qwt_ref, qb_ref, o_ref, t_ref):
    w = w_ref[:, 0:1]
    w = jnp.where(w == 0.0, 1.0, w)
    y = jnp.dot(h_ref[...], a_ref[...], preferred_element_type=jnp.float32)
    agg = jnp.dot(lo_ref[...], b1_ref[...], preferred_element_type=jnp.float32)
    agg = agg + jnp.dot(hi_ref[...], b2_ref[...],
                        preferred_element_type=jnp.float32)
    y = y + agg / w + wb_ref[...]
    y = jnp.where(y > 0, y, 0.01 * y)
    nrm = jnp.sqrt(jnp.sum(y * y, axis=1, keepdims=True))
    nrm = jnp.where(nrm == 0.0, 1.0, nrm)
    y = y / nrm
    o_ref[...] = y
    t = jnp.dot(y, qwt_ref[...], preferred_element_type=jnp.float32)
    t = t + qb_ref[...]
    t = jnp.where(t > 0, t, 0.01 * t)
    t_ref[0] = t[:, :HALF]
    t_ref[1] = t[:, HALF:]


def _apply_project(h, lo, hi, w16, a, b1, b2, wb2, qwt, qb2):
    return pl.pallas_call(
        _apply_proj_body,
        grid=(N // BN,),
        in_specs=[
            pl.BlockSpec((BN, F), lambda i: (i, 0)),
            pl.BlockSpec((BN, HALF), lambda i: (i, 0)),
            pl.BlockSpec((BN, HALF), lambda i: (i, 0)),
            pl.BlockSpec((BN, L), lambda i: (i, 0)),
            pl.BlockSpec((F, F), lambda i: (0, 0)),
            pl.BlockSpec((HALF, F), lambda i: (0, 0)),
            pl.BlockSpec((HALF, F), lambda i: (0, 0)),
            pl.BlockSpec((1, F), lambda i: (0, 0)),
            pl.BlockSpec((F, F), lambda i: (0, 0)),
            pl.BlockSpec((1, F), lambda i: (0, 0)),
        ],
        out_specs=[
            pl.BlockSpec((BN, F), lambda i: (i, 0)),
            pl.BlockSpec((2, BN, HALF), lambda i: (0, i, 0)),
        ],
        out_shape=[
            jax.ShapeDtypeStruct((N, F), jnp.float32),
            jax.ShapeDtypeStruct((2, N, HALF), jnp.float32),
        ],
    )(h, lo, hi, w16, a, b1, b2, wb2, qwt, qb2)


def _agg_body(table, src2, dst, ppr, out_lo, out_hi, out_w, srcb, dstb,
              sbidx, gbuf, pprb, wbuf, acc, wsh, gsem0, gsem1, gsem2, ssem0,
              ssem1, ssem2, psem0, psem1, psem2, wsem0, wsem1, wsem2):
    c = lax.axis_index("c")
    s = lax.axis_index("s")
    ebase = pl.multiple_of(c * E + s * EPT, 8)
    rbase = pl.multiple_of(s * RPT, 8)

    gsem = (gsem0, gsem1, gsem2)
    ssem = (ssem0, ssem1, ssem2)
    psem = (psem0, psem1, psem2)
    wsem = (wsem0, wsem1, wsem2)

    def _zb(i, carry):
        for k in range(HALF // L):
            gbuf[0, i, pl.ds(L * k, L)] = jnp.zeros((L,), jnp.float32)
        wbuf[0, i, :] = jnp.zeros((L,), jnp.float32)
        return carry

    lax.fori_loop(0, CHUNK, _zb, 0)
    for t in range(RPT // CHUNK):
        sl = pl.ds(rbase + t * CHUNK, CHUNK)
        pltpu.sync_copy(gbuf.at[0], acc.at[sl])

        @pl.when(c == 0)
        def _():
            pltpu.sync_copy(wbuf.at[0], wsh.at[sl])

    plsc.subcore_barrier()

    def _idx_issue(j, b):
        base = pl.multiple_of(ebase + j * CHUNK, 8)
        pltpu.async_copy(src2.at[pl.ds(base, CHUNK)], srcb.at[b], psem[b])
        bd = pl.multiple_of((ebase - c * E) + j * CHUNK, 8)
        pltpu.async_copy(dst.at[pl.ds(bd, CHUNK)], dstb.at[b], psem[b])
        pltpu.async_copy(ppr.at[pl.ds(bd, CHUNK)], pprb.at[b], psem[b])

    def _gather_issue(b):
        pltpu.make_async_copy(src2.at[pl.ds(0, CHUNK)], srcb.at[b],
                              psem[b]).wait()
        pltpu.make_async_copy(dst.at[pl.ds(0, CHUNK)], dstb.at[b],
                              psem[b]).wait()
        pltpu.make_async_copy(ppr.at[pl.ds(0, CHUNK)], pprb.at[b],
                              psem[b]).wait()
        pltpu.async_copy(table.at[srcb.at[b]], gbuf.at[b], gsem[b])

    def _wait_scatter(b):
        pltpu.make_async_copy(gbuf.at[b], acc.at[sbidx.at[b]], ssem[b]).wait()

        @pl.when(c == 0)
        def _():
            pltpu.make_async_copy(wbuf.at[b], wsh.at[sbidx.at[b]],
                                  wsem[b]).wait()

    def _process(j, b, first=False, last=0):
        bp = (b + 2) % 3
        bn = (b + 1) % 3
        if not first:
            _wait_scatter(bn)
        if last < 3:
            _gather_issue(bn)
        pltpu.make_async_copy(table.at[srcb.at[b]], gbuf.at[b],
                              gsem[b]).wait()

        def _grp(g, rc):
            pv = pprb[b, pl.ds(g * L, L)]
            for r in range(L):
                spl = jnp.broadcast_to(pv[r], (L,))
                row = g * L + r
                for k in range(HALF // L):
                    sl = pl.ds(L * k, L)
                    gbuf[b, row, sl] = gbuf[b, row, sl] * spl
                wbuf[b, row, :] = spl
            return rc

        lax.fori_loop(0, CHUNK // L, _grp, 0)
        for k in range(CHUNK // L):
            sl = pl.ds(L * k, L)
            sbidx[b, sl] = dstb[b, sl]
        pltpu.async_copy(gbuf.at[b], acc.at[sbidx.at[b]], ssem[b], add=True)

        @pl.when(c == 0)
        def _():
            pltpu.async_copy(wbuf.at[b], wsh.at[sbidx.at[b]], wsem[b],
                             add=True)

        if last < 2:
            _idx_issue(j + 2, bp)

    _idx_issue(0, 0)
    _idx_issue(1, 1)
    _gather_issue(0)
    _process(0, 0, first=True)
    _process(1, 1, first=True)
    _process(2, 2)

    def _trip(t, carry):
        j = 3 * t
        _process(j, 0)
        _process(j + 1, 1)
        _process(j + 2, 2)
        return carry

    lax.fori_loop(1, (NCHUNK - 2) // 3, _trip, 0)
    _process(NCHUNK - 2, 0, last=2)
    _process(NCHUNK - 1, 1, last=3)
    _wait_scatter(0)
    _wait_scatter(1)
    plsc.subcore_barrier()

    @pl.when(c == 0)
    def _():
        pltpu.sync_copy(acc.at[pl.ds(rbase, RPT)], out_lo.at[pl.ds(rbase, RPT)])
        pltpu.sync_copy(wsh.at[pl.ds(rbase, RPT)], out_w.at[pl.ds(rbase, RPT)])

    @pl.when(c == 1)
    def _():
        pltpu.sync_copy(acc.at[pl.ds(rbase, RPT)], out_hi.at[pl.ds(rbase, RPT)])


@functools.partial(jax.jit, static_argnames=())
def _aggregate(table, src2, dst, ppr):
    mesh = plsc.VectorSubcoreMesh(core_axis_name="c", subcore_axis_name="s")
    return pl.kernel(
        _agg_body,
        out_type=[
            jax.ShapeDtypeStruct((NPAD, HALF), jnp.float32),
            jax.ShapeDtypeStruct((NPAD, HALF), jnp.float32),
            jax.ShapeDtypeStruct((NPAD, L), jnp.float32),
        ],
        mesh=mesh,
        scratch_types=[
            pltpu.VMEM((3, CHUNK), jnp.int32),
            pltpu.VMEM((3, CHUNK), jnp.int32),
            pltpu.VMEM((3, CHUNK), jnp.int32),
            pltpu.VMEM((3, CHUNK, HALF), jnp.float32),
            pltpu.VMEM((3, CHUNK), jnp.float32),
            pltpu.VMEM((3, CHUNK, L), jnp.float32),
            pltpu.VMEM_SHARED((NPAD, HALF), jnp.float32),
            pltpu.VMEM_SHARED((NPAD, L), jnp.float32),
        ] + [pltpu.SemaphoreType.DMA] * 12,
        compiler_params=pltpu.CompilerParams(use_tc_tiling_on_sc=False),
    )(table, src2, dst, ppr)


def kernel(x, edge_index, ppr_weight, Q0_w, Q0_b, W0_w, W0_b, Q1_w, Q1_b,
           W1_w, W1_b):
    src = edge_index[0]
    dst = edge_index[1]
    src2 = jnp.concatenate([src, src + N])

    def wparts(ww, wb):
        wwt = ww.T
        return wwt[:F], wwt[F:F + HALF], wwt[F + HALF:], wb.reshape(1, F)

    a0, b10, b20, wb0 = wparts(W0_w, W0_b)
    a1, b11, b21, wb1 = wparts(W1_w, W1_b)

    table = _project(x, Q0_w.T, Q0_b.reshape(1, F))
    lo, hi, w16 = _aggregate(table, src2, dst, ppr_weight)
    h1, table3 = _apply_project(x, lo, hi, w16, a0, b10, b20, wb0,
                                Q1_w.T, Q1_b.reshape(1, F))
    lo, hi, w16 = _aggregate(table3.reshape(2 * N, HALF), src2, dst,
                             ppr_weight)
    return _apply(h1, lo, hi, w16, a1, b11, b21, wb1)
